# Initial kernel scaffold; baseline (speedup 1.0000x reference)
import jax
import jax.numpy as jnp
from jax.experimental import pallas as pl


def kernel(x, W_in, b_in, prompts, Ws, bs, Wq, bq, Wr, br, W1, b1, W2, b2, codebooks):
    raise NotImplementedError("write your pallas kernel here")


# speedup vs baseline: 3.8949x; 3.8949x over previous
import jax
import jax.numpy as jnp
from jax.experimental import pallas as pl
from jax.experimental.pallas import tpu as pltpu

_B = 32
_S = 256
_LAT = 512
_SYM = 128
_NODES = 4
_CODES = 512


def _dotT(a, b):
    return jax.lax.dot_general(a, b, (((1,), (1,)), ((), ())))


def _bus_kernel(x_ref, prompts_ref, Win_ref, bin_ref, Ws_ref, bs_ref, Wq_ref,
                bq_ref, Wr_ref, br_ref, W1_ref, b1_ref, W2_ref, b2_ref,
                cb_ref, out_ref):
    x = x_ref[0]
    ts = jnp.dot(x, Win_ref[...]) + bin_ref[...] + prompts_ref[0]

    token_state = ts
    outs = [ts]
    syms = [None]

    for i in range(1, _NODES):
        wq_row = Wq_ref[i:i + 1, :]
        bq_i = bq_ref[i:i + 1, :]
        rs = [jnp.zeros((_S, 1), jnp.float32) + bq_i]
        for t in range(1, i):
            rs.append(jnp.sum(syms[t] * wq_row, axis=1, keepdims=True) + bq_i)

        if i == 1:
            chosen = outs[0]
        elif i == 2:
            chosen = jnp.where(rs[1] > rs[0], outs[1], outs[0])
        else:
            pick0 = (rs[0] >= rs[1]) & (rs[0] >= rs[2])
            pick1 = rs[1] >= rs[2]
            chosen = jnp.where(pick0, outs[0],
                               jnp.where(pick1, outs[1], outs[2]))

        z_read = (jnp.dot(token_state, Wr_ref[i, :_LAT, :])
                  + jnp.dot(chosen, Wr_ref[i, _LAT:, :])
                  + br_ref[i:i + 1, :])

        raw = jnp.dot(z_read, Ws_ref[i]) + bs_ref[i:i + 1, :]

        cb = cb_ref[i]
        fsq = jnp.sum(raw * raw, axis=1, keepdims=True)
        cn = jnp.sum(cb * cb, axis=1)[None, :]
        d2 = (fsq - 2.0 * _dotT(raw, cb)) + cn
        idx = jnp.argmin(d2, axis=1)
        onehot = (idx[:, None]
                  == jax.lax.broadcasted_iota(jnp.int32, (_S, _CODES), 1)
                  ).astype(jnp.float32)
        q = jnp.dot(onehot, cb)

        h1 = jnp.maximum(
            jnp.dot(z_read, W1_ref[i, :_LAT, :])
            + jnp.dot(q, W1_ref[i, _LAT:, :])
            + b1_ref[i:i + 1, :], 0.0)
        node_out = (jnp.dot(h1, W2_ref[i]) + b2_ref[i:i + 1, :]
                    + token_state)

        outs.append(node_out)
        syms.append(q)
        token_state = node_out

    out_ref[0] = token_state


def kernel(x, W_in, b_in, prompts, Ws, bs, Wq, bq, Wr, br, W1, b1, W2, b2,
           codebooks):
    b_in2 = b_in.reshape(1, _LAT)
    Wq2 = Wq.reshape(_NODES, _SYM)
    bq2 = bq.reshape(_NODES, 1)

    def const(shape):
        return pl.BlockSpec(shape, lambda i: (0,) * len(shape))

    out = pl.pallas_call(
        _bus_kernel,
        grid=(_B,),
        in_specs=[
            pl.BlockSpec((1, _S, _LAT), lambda i: (i, 0, 0)),
            const((1, _S, _LAT)),
            const((_LAT, _LAT)),
            const((1, _LAT)),
            const((_NODES, _LAT, _SYM)),
            const((_NODES, _SYM)),
            const((_NODES, _SYM)),
            const((_NODES, 1)),
            const((_NODES, 2 * _LAT, _LAT)),
            const((_NODES, _LAT)),
            const((_NODES, _LAT + _SYM, _LAT)),
            const((_NODES, _LAT)),
            const((_NODES, _LAT, _LAT)),
            const((_NODES, _LAT)),
            const((_NODES, _CODES, _SYM)),
        ],
        out_specs=pl.BlockSpec((1, _S, _LAT), lambda i: (i, 0, 0)),
        out_shape=jax.ShapeDtypeStruct((_B, _S, _LAT), jnp.float32),
    )(x, prompts, W_in, b_in2, Ws, bs, Wq2, bq2, Wr, br, W1, b1, W2, b2,
      codebooks)
    return out


# speedup vs baseline: 6.6613x; 1.7103x over previous
import jax
import jax.numpy as jnp
from jax.experimental import pallas as pl
from jax.experimental.pallas import tpu as pltpu

_B = 32
_S = 256
_LAT = 512
_SYM = 128
_NODES = 4
_CODES = 512


def _dotT(a, b):
    return jax.lax.dot_general(a, b, (((1,), (1,)), ((), ())))


def _bf(v):
    return v.astype(jnp.bfloat16)


def _dot16(a, b):
    return jnp.dot(_bf(a), b, preferred_element_type=jnp.float32)


_NCH = 4


def _zip2(f):
    return [f(k) for k in range(_NCH)]


def _bus_kernel(x_ref, prompts_ref, Win_ref, bin_ref, Ws_ref, bs_ref,
                Wq_ref, bq_ref, Wr_ref, Wr1_ref, br_ref, W1_ref, b1_ref,
                W2_ref, b2_ref, cb_ref, cbh_ref, cbl_ref, W13_ref, W23_ref,
                out_ref):

    def quantize2(z2, Ws_i, bs_i, cb, cb_hi, cb_lo):
        raw2 = _zip2(lambda k: jnp.dot(z2[k], Ws_i) + bs_i)
        fsq2 = _zip2(lambda k: jnp.sum(raw2[k] * raw2[k], axis=1,
                                       keepdims=True))
        cn = jnp.sum(cb * cb, axis=1)[None, :]
        d2_2 = _zip2(lambda k: (fsq2[k] - 2.0 * _dotT(raw2[k], cb)) + cn)
        idx2 = _zip2(lambda k: jnp.argmin(d2_2[k], axis=1))
        oh2 = _zip2(lambda k: _bf(
            (idx2[k][:, None]
             == jax.lax.broadcasted_iota(jnp.int32, (_S, _CODES), 1)
             ).astype(jnp.float32)))
        return _zip2(lambda k:
                     jnp.dot(oh2[k], cb_hi,
                             preferred_element_type=jnp.float32)
                     + jnp.dot(oh2[k], cb_lo,
                               preferred_element_type=jnp.float32))

    ts2 = _zip2(lambda k: jnp.dot(x_ref[k], Win_ref[...]) + bin_ref[...]
                + prompts_ref[0])

    z1_2 = _zip2(lambda k: jnp.dot(ts2[k], Wr1_ref[...]) + br_ref[1:2, :])
    q1_2 = quantize2(z1_2, Ws_ref[1], bs_ref[1:2, :], cb_ref[1],
                     cbh_ref[1], cbl_ref[1])
    h1_2 = _zip2(lambda k: jnp.maximum(
        jnp.dot(z1_2[k], W1_ref[1, :_LAT, :])
        + jnp.dot(q1_2[k], W1_ref[1, _LAT:, :]) + b1_ref[1:2, :], 0.0))
    out1_2 = _zip2(lambda k: jnp.dot(h1_2[k], W2_ref[1]) + b2_ref[1:2, :]
                   + ts2[k])

    outs2 = [ts2, out1_2]
    syms2 = [None, q1_2]
    state2 = out1_2

    for i in (2, 3):
        wq_row = Wq_ref[i:i + 1, :]
        bq_i = bq_ref[i:i + 1, :]
        rs2 = [_zip2(lambda k: jnp.zeros((_S, 1), jnp.float32) + bq_i)]
        for t in range(1, i):
            rs2.append(_zip2(lambda k: jnp.sum(syms2[t][k] * wq_row,
                                               axis=1, keepdims=True)
                             + bq_i))

        if i == 2:
            chosen2 = _zip2(lambda k: jnp.where(rs2[1][k] > rs2[0][k],
                                                outs2[1][k], outs2[0][k]))
        else:
            chosen2 = _zip2(lambda k: jnp.where(
                (rs2[0][k] >= rs2[1][k]) & (rs2[0][k] >= rs2[2][k]),
                outs2[0][k],
                jnp.where(rs2[1][k] >= rs2[2][k], outs2[1][k],
                          outs2[2][k])))

        z2 = _zip2(lambda k: jnp.dot(state2[k], Wr_ref[i, :_LAT, :])
                   + jnp.dot(chosen2[k], Wr_ref[i, _LAT:, :])
                   + br_ref[i:i + 1, :])
        q2 = quantize2(z2, Ws_ref[i], bs_ref[i:i + 1, :], cb_ref[i],
                       cbh_ref[i], cbl_ref[i])

        if i == 3:
            h2 = _zip2(lambda k: jnp.maximum(
                _dot16(z2[k], W13_ref[:_LAT, :])
                + _dot16(q2[k], W13_ref[_LAT:, :]) + b1_ref[i:i + 1, :],
                0.0))
            node_out2 = _zip2(lambda k: _dot16(h2[k], W23_ref[...])
                              + b2_ref[i:i + 1, :] + state2[k])
        else:
            h2 = _zip2(lambda k: jnp.maximum(
                jnp.dot(z2[k], W1_ref[i, :_LAT, :])
                + jnp.dot(q2[k], W1_ref[i, _LAT:, :]) + b1_ref[i:i + 1, :],
                0.0))
            node_out2 = _zip2(lambda k: jnp.dot(h2[k], W2_ref[i])
                              + b2_ref[i:i + 1, :] + state2[k])

        outs2.append(node_out2)
        syms2.append(q2)
        state2 = node_out2

    for k in range(_NCH):
        out_ref[k] = state2[k]


def kernel(x, W_in, b_in, prompts, Ws, bs, Wq, bq, Wr, br, W1, b1, W2, b2,
           codebooks):
    b_in2 = b_in.reshape(1, _LAT)
    Wq2 = Wq.reshape(_NODES, _SYM)
    bq2 = bq.reshape(_NODES, 1)
    Wr1f = Wr[1, :_LAT, :] + Wr[1, _LAT:, :]
    cb_hi = codebooks.astype(jnp.bfloat16)
    cb_lo = (codebooks - cb_hi.astype(jnp.float32)).astype(jnp.bfloat16)
    W13 = W1[3].astype(jnp.bfloat16)
    W23 = W2[3].astype(jnp.bfloat16)

    def const(shape):
        return pl.BlockSpec(shape, lambda i: (0,) * len(shape))

    out = pl.pallas_call(
        _bus_kernel,
        grid=(_B // _NCH,),
        in_specs=[
            pl.BlockSpec((_NCH, _S, _LAT), lambda i: (i, 0, 0)),
            const((1, _S, _LAT)),
            const((_LAT, _LAT)),
            const((1, _LAT)),
            const((_NODES, _LAT, _SYM)),
            const((_NODES, _SYM)),
            const((_NODES, _SYM)),
            const((_NODES, 1)),
            const((_NODES, 2 * _LAT, _LAT)),
            const((_LAT, _LAT)),
            const((_NODES, _LAT)),
            const((_NODES, _LAT + _SYM, _LAT)),
            const((_NODES, _LAT)),
            const((_NODES, _LAT, _LAT)),
            const((_NODES, _LAT)),
            const((_NODES, _CODES, _SYM)),
            const((_NODES, _CODES, _SYM)),
            const((_NODES, _CODES, _SYM)),
            const((_LAT + _SYM, _LAT)),
            const((_LAT, _LAT)),
        ],
        out_specs=pl.BlockSpec((_NCH, _S, _LAT), lambda i: (i, 0, 0)),
        out_shape=jax.ShapeDtypeStruct((_B, _S, _LAT), jnp.float32),
    )(x, prompts, W_in, b_in2, Ws, bs, Wq2, bq2, Wr, Wr1f, br, W1, b1, W2,
      b2, codebooks, cb_hi, cb_lo, W13, W23)
    return out
